# baseline (speedup 1.0000x reference)
import functools

import jax
import jax.numpy as jnp
from jax import lax
from jax.experimental import pallas as pl
from jax.experimental.pallas import tpu as pltpu
from jax.experimental.pallas import tpu_sc as plsc

B = 4096
F = 100
E = 32
G = 26
S = 4
D = F * E
G_SC = 13
NW = 32
SB_ROWS = 512
NBUF = 3
BT = 256


def _sc_gather(x3, ms_flat):
    n_sc = G_SC * B * S
    rows_w = n_sc // NW
    nsb = rows_w // SB_ROWS
    mesh = plsc.VectorSubcoreMesh(core_axis_name="c", subcore_axis_name="s")

    @functools.partial(
        pl.kernel,
        out_type=jax.ShapeDtypeStruct((G * B * S, E), jnp.float32),
        mesh=mesh,
        scratch_types=(
            [pltpu.VMEM((2 * G * S,), jnp.int32)]
            + [pltpu.VMEM((SB_ROWS,), jnp.int32)] * NBUF
            + [pltpu.VMEM((SB_ROWS, E), jnp.float32)] * NBUF
            + [pltpu.SemaphoreType.DMA] * (2 * NBUF)
        ),
        compiler_params=pltpu.CompilerParams(
            use_tc_tiling_on_sc=False, needs_layout_passes=False),
    )
    def k(x_hbm, ms_hbm, out_hbm, ms_v, *bufs):
        idx = list(bufs[0:NBUF])
        data = list(bufs[NBUF:2 * NBUF])
        gsem = list(bufs[2 * NBUF:3 * NBUF])
        wsem = list(bufs[3 * NBUF:4 * NBUF])
        wid = lax.axis_index("s") * 2 + lax.axis_index("c")
        row0 = wid * rows_w
        pltpu.sync_copy(ms_hbm, ms_v)
        lanes = lax.iota(jnp.int32, 16)
        four = jnp.full((16,), S, jnp.int32)
        evec = jnp.full((16,), E, jnp.int32)
        s_lane = lax.rem(lanes, four)
        q_lane = lax.div(lanes, four)

        def compute_idx(sb, buf):
            sb_base = row0 + sb * SB_ROWS
            g = lax.div(sb_base, B * S)
            b0 = lax.div(lax.rem(sb_base, B * S), S)
            fvec = lax.div(
                plsc.load_gather(
                    ms_v, [jnp.full((16,), g * (2 * S), jnp.int32)
                           + s_lane * 2]),
                evec)
            base = (jnp.full((16,), b0, jnp.int32) + q_lane) * F + fvec

            def body(kk, _):
                idx[buf][pl.ds(kk * 16, 16)] = (
                    base + jnp.full((16,), kk * (S * F), jnp.int32))
                return 0

            lax.fori_loop(0, SB_ROWS // 16, body, 0)

        def issue_gather(buf):
            return pltpu.async_copy(x_hbm.at[idx[buf]], data[buf], gsem[buf])

        def issue_write(sb, buf):
            return pltpu.async_copy(
                data[buf], out_hbm.at[pl.ds(row0 + sb * SB_ROWS, SB_ROWS)],
                wsem[buf])

        gd = [None] * nsb
        wd = [None] * nsb
        for sb in range(nsb):
            buf = sb % NBUF
            if sb >= NBUF:
                wd[sb - NBUF].wait()
            compute_idx(sb, buf)
            gd[sb] = issue_gather(buf)
            if sb >= 1:
                gd[sb - 1].wait()
                wd[sb - 1] = issue_write(sb - 1, (sb - 1) % NBUF)
        gd[nsb - 1].wait()
        wd[nsb - 1] = issue_write(nsb - 1, (nsb - 1) % NBUF)
        for sb in range(nsb - NBUF + 1, nsb):
            wd[sb].wait()

    return k(x3, ms_flat)


def _tc_slices(x, starts):
    g_tc = G - G_SC

    def body(ms_ref, x_ref, out_ref):
        for g in range(g_tc):
            for s in range(S):
                st = ms_ref[g, s]
                st128 = pl.multiple_of((st // 128) * 128, 128)
                r = st - st128
                w = x_ref[:, pl.ds(st128, 128)]
                out_ref[g, :, s * E:(s + 1) * E] = (
                    pltpu.roll(w, -r, 1)[:, :E])

    return pl.pallas_call(
        body,
        grid=(B // BT,),
        in_specs=[
            pl.BlockSpec(memory_space=pltpu.SMEM),
            pl.BlockSpec((BT, D), lambda i: (i, 0)),
        ],
        out_specs=pl.BlockSpec((g_tc, BT, S * E), lambda i: (0, i, 0)),
        out_shape=jax.ShapeDtypeStruct((g_tc, B, S * E), jnp.float32),
    )(starts, x)


def kernel(input_tensor, many_slices):
    ms = jnp.asarray(many_slices).astype(jnp.int32)
    x3 = input_tensor.reshape(B * F, E)
    sc_out = _sc_gather(x3, ms.reshape(-1)).reshape(G, B, S * E)
    tc_part = _tc_slices(input_tensor, ms[G_SC:, :, 0])
    return lax.dynamic_update_slice(sc_out, tc_part, (G_SC, 0, 0))


# speedup vs baseline: 1.5415x; 1.5415x over previous
import functools

import jax
import jax.numpy as jnp
from jax import lax
from jax.experimental import pallas as pl
from jax.experimental.pallas import tpu as pltpu
from jax.experimental.pallas import tpu_sc as plsc

B = 4096
F = 100
E = 32
G = 26
S = 4
N = G * B * S
NW = 32
ROWS_W = N // NW
SB_ROWS = 1024
NSB = ROWS_W // SB_ROWS
NBUF = 3


def _sc_gather(x3, ms_flat):
    mesh = plsc.VectorSubcoreMesh(core_axis_name="c", subcore_axis_name="s")

    @functools.partial(
        pl.kernel,
        out_type=jax.ShapeDtypeStruct((N, E), jnp.float32),
        mesh=mesh,
        scratch_types=(
            [pltpu.VMEM((2 * G * S,), jnp.int32)]
            + [pltpu.VMEM((SB_ROWS,), jnp.int32)] * NBUF
            + [pltpu.VMEM((SB_ROWS, E), jnp.float32)] * NBUF
            + [pltpu.SemaphoreType.DMA] * (2 * NBUF)
        ),
        compiler_params=pltpu.CompilerParams(
            use_tc_tiling_on_sc=False, needs_layout_passes=False),
    )
    def k(x_hbm, ms_hbm, out_hbm, ms_v, *bufs):
        idx = list(bufs[0:NBUF])
        data = list(bufs[NBUF:2 * NBUF])
        gsem = list(bufs[2 * NBUF:3 * NBUF])
        wsem = list(bufs[3 * NBUF:4 * NBUF])
        wid = lax.axis_index("s") * 2 + lax.axis_index("c")
        row0 = wid * ROWS_W
        pltpu.sync_copy(ms_hbm, ms_v)
        lanes = lax.iota(jnp.int32, 16)
        four = jnp.full((16,), S, jnp.int32)
        evec = jnp.full((16,), E, jnp.int32)
        s_lane = lax.rem(lanes, four)
        q_lane = lax.div(lanes, four)

        def compute_idx(sb, buf):
            sb_base = row0 + sb * SB_ROWS
            g = lax.div(sb_base, B * S)
            b0 = lax.div(lax.rem(sb_base, B * S), S)
            fvec = lax.div(
                plsc.load_gather(
                    ms_v, [jnp.full((16,), g * (2 * S), jnp.int32)
                           + s_lane * 2]),
                evec)
            base = (jnp.full((16,), b0, jnp.int32) + q_lane) * F + fvec

            def body(kk, _):
                idx[buf][pl.ds(kk * 16, 16)] = (
                    base + jnp.full((16,), kk * (S * F), jnp.int32))
                return 0

            lax.fori_loop(0, SB_ROWS // 16, body, 0)

        def issue_gather(buf):
            return pltpu.async_copy(x_hbm.at[idx[buf]], data[buf], gsem[buf])

        def issue_write(sb, buf):
            return pltpu.async_copy(
                data[buf], out_hbm.at[pl.ds(row0 + sb * SB_ROWS, SB_ROWS)],
                wsem[buf])

        gd = [None] * NSB
        wd = [None] * NSB
        for sb in range(NSB):
            buf = sb % NBUF
            if sb >= NBUF:
                wd[sb - NBUF].wait()
            compute_idx(sb, buf)
            gd[sb] = issue_gather(buf)
            if sb >= 1:
                gd[sb - 1].wait()
                wd[sb - 1] = issue_write(sb - 1, (sb - 1) % NBUF)
        gd[NSB - 1].wait()
        wd[NSB - 1] = issue_write(NSB - 1, (NSB - 1) % NBUF)
        for sb in range(NSB - NBUF + 1, NSB):
            wd[sb].wait()

    return k(x3, ms_flat)


def kernel(input_tensor, many_slices):
    x3 = input_tensor.reshape(B * F, E)
    ms_flat = jnp.asarray(many_slices).astype(jnp.int32).reshape(-1)
    out = _sc_gather(x3, ms_flat)
    return out.reshape(G, B, S * E)


# speedup vs baseline: 1.5575x; 1.0104x over previous
import functools

import jax
import jax.numpy as jnp
from jax import lax
from jax.experimental import pallas as pl
from jax.experimental.pallas import tpu as pltpu
from jax.experimental.pallas import tpu_sc as plsc

B = 4096
F = 100
E = 32
G = 26
S = 4
N = G * B * S
NW = 32
ROWS_W = N // NW
SB_ROWS = 1024
NSB = ROWS_W // SB_ROWS
NBUF = 3


def _sc_gather(x3, ms_flat):
    mesh = plsc.VectorSubcoreMesh(core_axis_name="c", subcore_axis_name="s")

    @functools.partial(
        pl.kernel,
        out_type=jax.ShapeDtypeStruct((N, E), jnp.float32),
        mesh=mesh,
        scratch_types=(
            [pltpu.VMEM((2 * G * S,), jnp.int32)]
            + [pltpu.VMEM((SB_ROWS,), jnp.int32)] * NBUF
            + [pltpu.VMEM((SB_ROWS, E), jnp.float32)] * NBUF
            + [pltpu.SemaphoreType.DMA] * (2 * NBUF)
        ),
        compiler_params=pltpu.CompilerParams(
            use_tc_tiling_on_sc=False, needs_layout_passes=False),
    )
    def k(x_hbm, ms_hbm, out_hbm, ms_v, *bufs):
        idx = list(bufs[0:NBUF])
        data = list(bufs[NBUF:2 * NBUF])
        gsem = list(bufs[2 * NBUF:3 * NBUF])
        wsem = list(bufs[3 * NBUF:4 * NBUF])
        wid = lax.axis_index("s") * 2 + lax.axis_index("c")
        row0 = wid * ROWS_W
        pltpu.sync_copy(ms_hbm, ms_v)
        lanes = lax.iota(jnp.int32, 16)
        four = jnp.full((16,), S, jnp.int32)
        evec = jnp.full((16,), E, jnp.int32)
        s_lane = lax.rem(lanes, four)
        q_lane = lax.div(lanes, four)

        def compute_idx(sb, buf):
            sb_base = row0 + sb * SB_ROWS
            g = lax.div(sb_base, B * S)
            b0 = lax.div(lax.rem(sb_base, B * S), S)
            fvec = lax.div(
                plsc.load_gather(
                    ms_v, [jnp.full((16,), g * (2 * S), jnp.int32)
                           + s_lane * 2]),
                evec)
            base = (jnp.full((16,), b0, jnp.int32) + q_lane) * F + fvec

            def body(kk, _):
                idx[buf][pl.ds(kk * 16, 16)] = (
                    base + jnp.full((16,), kk * (S * F), jnp.int32))
                return 0

            lax.fori_loop(0, SB_ROWS // 16, body, 0)

        def issue_gather(buf):
            return pltpu.async_copy(x_hbm.at[idx[buf]], data[buf], gsem[buf])

        def issue_write(sb, buf):
            return pltpu.async_copy(
                data[buf], out_hbm.at[pl.ds(row0 + sb * SB_ROWS, SB_ROWS)],
                wsem[buf])

        gd = [None] * NSB
        wd = [None] * NSB
        compute_idx(0, 0)
        plsc.subcore_barrier()
        for sb in range(NSB):
            buf = sb % NBUF
            if sb >= NBUF:
                wd[sb - NBUF].wait()
            gd[sb] = issue_gather(buf)
            if sb + 1 < NSB:
                compute_idx(sb + 1, (sb + 1) % NBUF)
            if sb >= 1:
                gd[sb - 1].wait()
                wd[sb - 1] = issue_write(sb - 1, (sb - 1) % NBUF)
        gd[NSB - 1].wait()
        wd[NSB - 1] = issue_write(NSB - 1, (NSB - 1) % NBUF)
        for sb in range(NSB - NBUF, NSB):
            wd[sb].wait()

    return k(x3, ms_flat)


def kernel(input_tensor, many_slices):
    x3 = input_tensor.reshape(B * F, E)
    ms_flat = jnp.asarray(many_slices).astype(jnp.int32).reshape(-1)
    out = _sc_gather(x3, ms_flat)
    return out.reshape(G, B, S * E)
